# baseline (speedup 1.0000x reference)
import functools

import jax
import jax.numpy as jnp
from jax import lax
from jax.experimental import pallas as pl
from jax.experimental.pallas import tpu as pltpu
from jax.experimental.pallas import tpu_sc as plsc

B, C, N, K, OUT = 1, 128, 10000, 32, 128
NC, NS, L = 2, 16, 16
NW = NC * NS
FPW = OUT // NW
PPW = FPW // 2
N_PAD = 10240
NB = 1024
NBLK = N_PAD // NB
OH = OUT // 2


def _pack_rows(even, odd):
    lo = lax.bitcast_convert_type(even.astype(jnp.bfloat16), jnp.uint16)
    hi = lax.bitcast_convert_type(odd.astype(jnp.bfloat16), jnp.uint16)
    return lo.astype(jnp.int32) | (hi.astype(jnp.int32) << 16)


def _tc_tables(x_ref, m1e_ref, m1o_ref, m2e_ref, m2o_ref, be_ref, bo_ref,
               a_ref, c_ref):
    xb = x_ref[...]
    f32 = jnp.float32
    ae = jnp.dot(m1e_ref[...], xb, preferred_element_type=f32) + be_ref[...]
    ao = jnp.dot(m1o_ref[...], xb, preferred_element_type=f32) + bo_ref[...]
    ce = jnp.dot(m2e_ref[...], xb, preferred_element_type=f32)
    co = jnp.dot(m2o_ref[...], xb, preferred_element_type=f32)
    a_ref[...] = _pack_rows(ae, ao)
    c_ref[...] = _pack_rows(ce, co)


def _make_tables(x2, w, b):
    bn = 1280
    grid = N_PAD // bn
    w1 = w[:, :C] - w[:, C:]
    w2 = w[:, C:]
    mat_spec = pl.BlockSpec((OH, C), lambda i: (0, 0))
    vec_spec = pl.BlockSpec((OH, 1), lambda i: (0, 0))
    return pl.pallas_call(
        _tc_tables,
        grid=(grid,),
        in_specs=[
            pl.BlockSpec((C, bn), lambda i: (0, i)),
            mat_spec, mat_spec, mat_spec, mat_spec, vec_spec, vec_spec,
        ],
        out_specs=[
            pl.BlockSpec((OH, bn), lambda i: (0, i)),
            pl.BlockSpec((OH, bn), lambda i: (0, i)),
        ],
        out_shape=[
            jax.ShapeDtypeStruct((OH, N_PAD), jnp.int32),
            jax.ShapeDtypeStruct((OH, N_PAD), jnp.int32),
        ],
    )(
        x2, w1[0::2], w1[1::2], w2[0::2], w2[1::2],
        b[0::2].reshape(OH, 1), b[1::2].reshape(OH, 1),
    )


@functools.partial(
    pl.kernel,
    out_type=jax.ShapeDtypeStruct((NW, PPW, N_PAD), jnp.int32),
    mesh=plsc.VectorSubcoreMesh(
        core_axis_name="c", subcore_axis_name="s", num_cores=NC, num_subcores=NS
    ),
    compiler_params=pltpu.CompilerParams(needs_layout_passes=False),
    scratch_types=[
        pltpu.VMEM((PPW * N_PAD,), jnp.int32),
        pltpu.VMEM((PPW * N_PAD,), jnp.int32),
        pltpu.VMEM((NB * K,), jnp.int32),
        pltpu.VMEM((NB * K,), jnp.int32),
        pltpu.VMEM((PPW, NB), jnp.int32),
        pltpu.VMEM((PPW, NB), jnp.int32),
        pltpu.SemaphoreType.DMA,
        pltpu.SemaphoreType.DMA,
        pltpu.SemaphoreType.DMA,
        pltpu.SemaphoreType.DMA,
    ],
)
def _sc_edge_max(
    a_hbm, c_hbm, ij_hbm, out_hbm,
    a_v, c_v, ij_v0, ij_v1, ob_v0, ob_v1, sem_in0, sem_in1, sem_out0, sem_out1,
):
    wid = lax.axis_index("c") * NS + lax.axis_index("s")
    pltpu.sync_copy(a_hbm.at[wid], a_v)
    pltpu.sync_copy(c_hbm.at[wid], c_v)

    ij_bufs = (ij_v0, ij_v1)
    ob_bufs = (ob_v0, ob_v1)
    sem_in = (sem_in0, sem_in1)
    sem_out = (sem_out0, sem_out1)
    p_off = [jnp.full((L,), p * N_PAD, jnp.int32) for p in range(PPW)]
    kiota = lax.iota(jnp.int32, L) * K

    def start_in(blk):
        bi = blk % 2
        return pltpu.async_copy(
            ij_hbm.at[pl.ds(blk * NB * K, NB * K)], ij_bufs[bi], sem_in[bi]
        )

    pending = {0: start_in(0)}
    out_pending = {}
    for blk in range(NBLK):
        bi = blk % 2
        if blk + 1 < NBLK:
            pending[blk + 1] = start_in(blk + 1)
        pending.pop(blk).wait()
        if blk - 2 in out_pending:
            out_pending.pop(blk - 2).wait()

        ij_b = ij_bufs[bi]
        ob_b = ob_bufs[bi]

        def nb_body(nb, _):
            base = nb * L

            def k_body(k, accs):
                w = plsc.load_gather(
                    ij_b, [jnp.full((L,), base * K + k, jnp.int32) + kiota]
                )
                iv = w & 0xFFFF
                jv = lax.shift_right_logical(w, 16)
                new = []
                for p in range(PPW):
                    av = plsc.bitcast(
                        plsc.load_gather(a_v, [p_off[p] + iv]), jnp.bfloat16
                    )
                    cv = plsc.bitcast(
                        plsc.load_gather(c_v, [p_off[p] + jv]), jnp.bfloat16
                    )
                    new.append(jnp.maximum(accs[p], av + cv))
                return tuple(new)

            accs = lax.fori_loop(
                0, K, k_body,
                tuple(jnp.zeros((2 * L,), jnp.bfloat16) for _ in range(PPW)),
            )
            for p in range(PPW):
                ob_b[p, pl.ds(base, L)] = plsc.bitcast(accs[p], jnp.int32)
            return 0

        lax.fori_loop(0, NB // L, nb_body, 0)
        out_pending[blk] = pltpu.async_copy(
            ob_b, out_hbm.at[wid, :, pl.ds(blk * NB, NB)], sem_out[bi]
        )
    for h in out_pending.values():
        h.wait()


def kernel(x, edge_index, W, b):
    x2 = x[0, :, :, 0]
    x2 = jnp.pad(x2, ((0, 0), (0, N_PAD - N)))
    a_t, c_t = _make_tables(x2, W, b)
    a_r = a_t.reshape(NW, PPW * N_PAD)
    c_r = c_t.reshape(NW, PPW * N_PAD)

    ij = edge_index[1, 0] | (edge_index[0, 0] << 16)
    ij = jnp.pad(ij, ((0, N_PAD - N), (0, 0))).reshape(N_PAD * K)

    out_r = _sc_edge_max(a_r, c_r, ij)

    lo = lax.bitcast_convert_type(
        (out_r & 0xFFFF).astype(jnp.uint16), jnp.bfloat16
    )
    hi = lax.bitcast_convert_type(
        lax.shift_right_logical(out_r, 16).astype(jnp.uint16), jnp.bfloat16
    )
    out = jnp.stack([lo, hi], axis=2)
    out = out.reshape(OUT, N_PAD)[:, :N].astype(jnp.float32)
    return out.reshape(1, OUT, N, 1)


# speedup vs baseline: 1.7962x; 1.7962x over previous
import functools

import jax
import jax.numpy as jnp
from jax import lax
from jax.experimental import pallas as pl
from jax.experimental.pallas import tpu as pltpu
from jax.experimental.pallas import tpu_sc as plsc

B, C, N, K, OUT = 1, 128, 10000, 32, 128
NC, NS, L = 2, 16, 16
NW = NC * NS
FPW = OUT // NW
PPW = FPW // 2
N_PAD = 10240
NB = 1024
NBLK = N_PAD // NB
OH = OUT // 2


def _pack_rows(even, odd):
    lo = lax.bitcast_convert_type(even.astype(jnp.bfloat16), jnp.uint16)
    hi = lax.bitcast_convert_type(odd.astype(jnp.bfloat16), jnp.uint16)
    return lo.astype(jnp.int32) | (hi.astype(jnp.int32) << 16)


def _tc_tables(x_ref, m1e_ref, m1o_ref, m2e_ref, m2o_ref, be_ref, bo_ref,
               a_ref, c_ref):
    xb = x_ref[...]
    f32 = jnp.float32
    ae = jnp.dot(m1e_ref[...], xb, preferred_element_type=f32) + be_ref[...]
    ao = jnp.dot(m1o_ref[...], xb, preferred_element_type=f32) + bo_ref[...]
    ce = jnp.dot(m2e_ref[...], xb, preferred_element_type=f32)
    co = jnp.dot(m2o_ref[...], xb, preferred_element_type=f32)
    a_ref[...] = _pack_rows(ae, ao)
    c_ref[...] = _pack_rows(ce, co)


def _make_tables(x2, w, b):
    bn = 1280
    grid = N_PAD // bn
    w1 = w[:, :C] - w[:, C:]
    w2 = w[:, C:]
    mat_spec = pl.BlockSpec((OH, C), lambda i: (0, 0))
    vec_spec = pl.BlockSpec((OH, 1), lambda i: (0, 0))
    return pl.pallas_call(
        _tc_tables,
        grid=(grid,),
        in_specs=[
            pl.BlockSpec((C, bn), lambda i: (0, i)),
            mat_spec, mat_spec, mat_spec, mat_spec, vec_spec, vec_spec,
        ],
        out_specs=[
            pl.BlockSpec((OH, bn), lambda i: (0, i)),
            pl.BlockSpec((OH, bn), lambda i: (0, i)),
        ],
        out_shape=[
            jax.ShapeDtypeStruct((OH, N_PAD), jnp.int32),
            jax.ShapeDtypeStruct((OH, N_PAD), jnp.int32),
        ],
    )(
        x2, w1[0::2], w1[1::2], w2[0::2], w2[1::2],
        b[0::2].reshape(OH, 1), b[1::2].reshape(OH, 1),
    )


@functools.partial(
    pl.kernel,
    out_type=jax.ShapeDtypeStruct((NW, PPW, N_PAD), jnp.int32),
    mesh=plsc.VectorSubcoreMesh(
        core_axis_name="c", subcore_axis_name="s", num_cores=NC, num_subcores=NS
    ),
    compiler_params=pltpu.CompilerParams(needs_layout_passes=False),
    scratch_types=[
        pltpu.VMEM((PPW * N_PAD,), jnp.int32),
        pltpu.VMEM((PPW * N_PAD,), jnp.int32),
        pltpu.VMEM((K, NB), jnp.int32),
        pltpu.VMEM((K, NB), jnp.int32),
        pltpu.VMEM((PPW, NB), jnp.int32),
        pltpu.VMEM((PPW, NB), jnp.int32),
        pltpu.SemaphoreType.DMA,
        pltpu.SemaphoreType.DMA,
        pltpu.SemaphoreType.DMA,
        pltpu.SemaphoreType.DMA,
    ],
)
def _sc_edge_max(
    a_hbm, c_hbm, ij_hbm, out_hbm,
    a_v, c_v, ij_v0, ij_v1, ob_v0, ob_v1, sem_in0, sem_in1, sem_out0, sem_out1,
):
    wid = lax.axis_index("c") * NS + lax.axis_index("s")
    pltpu.sync_copy(a_hbm.at[wid], a_v)
    pltpu.sync_copy(c_hbm.at[wid], c_v)

    ij_bufs = (ij_v0, ij_v1)
    ob_bufs = (ob_v0, ob_v1)
    sem_in = (sem_in0, sem_in1)
    sem_out = (sem_out0, sem_out1)
    p_off = [jnp.full((L,), p * N_PAD, jnp.int32) for p in range(PPW)]

    def start_in(blk):
        bi = blk % 2
        return pltpu.async_copy(
            ij_hbm.at[:, pl.ds(blk * NB, NB)], ij_bufs[bi], sem_in[bi]
        )

    pending = {0: start_in(0)}
    out_pending = {}
    for blk in range(NBLK):
        bi = blk % 2
        if blk + 1 < NBLK:
            pending[blk + 1] = start_in(blk + 1)
        pending.pop(blk).wait()
        if blk - 2 in out_pending:
            out_pending.pop(blk - 2).wait()

        ij_b = ij_bufs[bi]
        ob_b = ob_bufs[bi]

        def nb_body(nb, _):
            base = nb * L

            def k_body(k, accs):
                w = ij_b[k, pl.ds(base, L)]
                iv = w & 0xFFFF
                jv = lax.shift_right_logical(w, 16)
                new = []
                for p in range(PPW):
                    av = plsc.bitcast(
                        plsc.load_gather(a_v, [p_off[p] + iv]), jnp.bfloat16
                    )
                    cv = plsc.bitcast(
                        plsc.load_gather(c_v, [p_off[p] + jv]), jnp.bfloat16
                    )
                    new.append(jnp.maximum(accs[p], av + cv))
                return tuple(new)

            accs = lax.fori_loop(
                0, K, k_body,
                tuple(jnp.zeros((2 * L,), jnp.bfloat16) for _ in range(PPW)),
            )
            for p in range(PPW):
                ob_b[p, pl.ds(base, L)] = plsc.bitcast(accs[p], jnp.int32)
            return 0

        lax.fori_loop(0, NB // L, nb_body, 0)
        out_pending[blk] = pltpu.async_copy(
            ob_b, out_hbm.at[wid, :, pl.ds(blk * NB, NB)], sem_out[bi]
        )
    for h in out_pending.values():
        h.wait()


def kernel(x, edge_index, W, b):
    x2 = x[0, :, :, 0]
    x2 = jnp.pad(x2, ((0, 0), (0, N_PAD - N)))
    a_t, c_t = _make_tables(x2, W, b)
    a_r = a_t.reshape(NW, PPW * N_PAD)
    c_r = c_t.reshape(NW, PPW * N_PAD)

    ij = edge_index[1, 0] | (edge_index[0, 0] << 16)
    ij = jnp.pad(ij.T, ((0, 0), (0, N_PAD - N)))

    out_r = _sc_edge_max(a_r, c_r, ij)

    lo = lax.bitcast_convert_type(
        (out_r & 0xFFFF).astype(jnp.uint16), jnp.bfloat16
    )
    hi = lax.bitcast_convert_type(
        lax.shift_right_logical(out_r, 16).astype(jnp.uint16), jnp.bfloat16
    )
    out = jnp.stack([lo, hi], axis=2)
    out = out.reshape(OUT, N_PAD)[:, :N].astype(jnp.float32)
    return out.reshape(1, OUT, N, 1)


# speedup vs baseline: 1.9305x; 1.0748x over previous
import functools

import jax
import jax.numpy as jnp
from jax import lax
from jax.experimental import pallas as pl
from jax.experimental.pallas import tpu as pltpu
from jax.experimental.pallas import tpu_sc as plsc

B, C, N, K, OUT = 1, 128, 10000, 32, 128
NC, NS, L = 2, 16, 16
NW = NC * NS
FPW = OUT // NW
PPW = FPW // 2
N_PAD = 10240
NB = 1024
NBLK = N_PAD // NB
OH = OUT // 2


def _pack_rows(even, odd):
    lo = lax.bitcast_convert_type(even.astype(jnp.bfloat16), jnp.uint16)
    hi = lax.bitcast_convert_type(odd.astype(jnp.bfloat16), jnp.uint16)
    return lo.astype(jnp.int32) | (hi.astype(jnp.int32) << 16)


def _tc_tables(x_ref, m1e_ref, m1o_ref, m2e_ref, m2o_ref, be_ref, bo_ref,
               a_ref, c_ref):
    xb = x_ref[...]
    f32 = jnp.float32
    ae = jnp.dot(m1e_ref[...], xb, preferred_element_type=f32) + be_ref[...]
    ao = jnp.dot(m1o_ref[...], xb, preferred_element_type=f32) + bo_ref[...]
    ce = jnp.dot(m2e_ref[...], xb, preferred_element_type=f32)
    co = jnp.dot(m2o_ref[...], xb, preferred_element_type=f32)
    a_ref[...] = _pack_rows(ae, ao)
    c_ref[...] = _pack_rows(ce, co)


def _make_tables(x2, w, b):
    bn = 1280
    grid = N_PAD // bn
    w1 = w[:, :C] - w[:, C:]
    w2 = w[:, C:]
    mat_spec = pl.BlockSpec((OH, C), lambda i: (0, 0))
    vec_spec = pl.BlockSpec((OH, 1), lambda i: (0, 0))
    return pl.pallas_call(
        _tc_tables,
        grid=(grid,),
        in_specs=[
            pl.BlockSpec((C, bn), lambda i: (0, i)),
            mat_spec, mat_spec, mat_spec, mat_spec, vec_spec, vec_spec,
        ],
        out_specs=[
            pl.BlockSpec((OH, bn), lambda i: (0, i)),
            pl.BlockSpec((OH, bn), lambda i: (0, i)),
        ],
        out_shape=[
            jax.ShapeDtypeStruct((OH, N_PAD), jnp.int32),
            jax.ShapeDtypeStruct((OH, N_PAD), jnp.int32),
        ],
    )(
        x2, w1[0::2], w1[1::2], w2[0::2], w2[1::2],
        b[0::2].reshape(OH, 1), b[1::2].reshape(OH, 1),
    )


@functools.partial(
    pl.kernel,
    out_type=jax.ShapeDtypeStruct((NW, PPW, N_PAD), jnp.int32),
    mesh=plsc.VectorSubcoreMesh(
        core_axis_name="c", subcore_axis_name="s", num_cores=NC, num_subcores=NS
    ),
    compiler_params=pltpu.CompilerParams(needs_layout_passes=False),
    scratch_types=[
        pltpu.VMEM((PPW * N_PAD,), jnp.int32),
        pltpu.VMEM((PPW * N_PAD,), jnp.int32),
        pltpu.VMEM((K, NB), jnp.int32),
        pltpu.VMEM((K, NB), jnp.int32),
        pltpu.VMEM((PPW, NB), jnp.int32),
        pltpu.VMEM((PPW, NB), jnp.int32),
        pltpu.SemaphoreType.DMA,
        pltpu.SemaphoreType.DMA,
        pltpu.SemaphoreType.DMA,
        pltpu.SemaphoreType.DMA,
    ],
)
def _sc_edge_max(
    a_hbm, c_hbm, ij_hbm, out_hbm,
    a_v, c_v, ij_v0, ij_v1, ob_v0, ob_v1, sem_in0, sem_in1, sem_out0, sem_out1,
):
    wid = lax.axis_index("c") * NS + lax.axis_index("s")
    pltpu.sync_copy(a_hbm.at[wid], a_v)
    pltpu.sync_copy(c_hbm.at[wid], c_v)

    ij_bufs = (ij_v0, ij_v1)
    ob_bufs = (ob_v0, ob_v1)
    sem_in = (sem_in0, sem_in1)
    sem_out = (sem_out0, sem_out1)
    p_off = [jnp.full((L,), p * N_PAD, jnp.int32) for p in range(PPW)]

    def start_in(blk):
        bi = blk % 2
        return pltpu.async_copy(
            ij_hbm.at[:, pl.ds(blk * NB, NB)], ij_bufs[bi], sem_in[bi]
        )

    pending = {0: start_in(0)}
    out_pending = {}
    for blk in range(NBLK):
        bi = blk % 2
        if blk + 1 < NBLK:
            pending[blk + 1] = start_in(blk + 1)
        pending.pop(blk).wait()
        if blk - 2 in out_pending:
            out_pending.pop(blk - 2).wait()

        ij_b = ij_bufs[bi]
        ob_b = ob_bufs[bi]

        def nb_body(nb):
            base = nb * L

            def k8_body(k8, accs):
                accs = list(accs)
                for kk in range(8):
                    w = ij_b[k8 * 8 + kk, pl.ds(base, L)]
                    iv = w & 0xFFFF
                    jv = lax.shift_right_logical(w, 16)
                    chain = kk % 2
                    for p in range(PPW):
                        av = plsc.bitcast(
                            plsc.load_gather(a_v, [p_off[p] + iv]), jnp.bfloat16
                        )
                        cv = plsc.bitcast(
                            plsc.load_gather(c_v, [p_off[p] + jv]), jnp.bfloat16
                        )
                        s = 2 * p + chain
                        accs[s] = jnp.maximum(accs[s], av + cv)
                return tuple(accs)

            accs = lax.fori_loop(
                0, K // 8, k8_body,
                tuple(jnp.zeros((2 * L,), jnp.bfloat16) for _ in range(2 * PPW)),
            )
            for p in range(PPW):
                ob_b[p, pl.ds(base, L)] = plsc.bitcast(
                    jnp.maximum(accs[2 * p], accs[2 * p + 1]), jnp.int32
                )

        plsc.parallel_loop(0, NB // L, 1)(nb_body)
        out_pending[blk] = pltpu.async_copy(
            ob_b, out_hbm.at[wid, :, pl.ds(blk * NB, NB)], sem_out[bi]
        )
    for h in out_pending.values():
        h.wait()


def kernel(x, edge_index, W, b):
    x2 = x[0, :, :, 0]
    x2 = jnp.pad(x2, ((0, 0), (0, N_PAD - N)))
    a_t, c_t = _make_tables(x2, W, b)
    a_r = a_t.reshape(NW, PPW * N_PAD)
    c_r = c_t.reshape(NW, PPW * N_PAD)

    ij = edge_index[1, 0] | (edge_index[0, 0] << 16)
    ij = jnp.pad(ij.T, ((0, 0), (0, N_PAD - N)))

    out_r = _sc_edge_max(a_r, c_r, ij)

    lo = lax.bitcast_convert_type(
        (out_r & 0xFFFF).astype(jnp.uint16), jnp.bfloat16
    )
    hi = lax.bitcast_convert_type(
        lax.shift_right_logical(out_r, 16).astype(jnp.uint16), jnp.bfloat16
    )
    out = jnp.stack([lo, hi], axis=2)
    out = out.reshape(OUT, N_PAD)[:, :N].astype(jnp.float32)
    return out.reshape(1, OUT, N, 1)


# speedup vs baseline: 2.1460x; 1.1116x over previous
import functools

import jax
import jax.numpy as jnp
from jax import lax
from jax.experimental import pallas as pl
from jax.experimental.pallas import tpu as pltpu
from jax.experimental.pallas import tpu_sc as plsc

B, C, N, K, OUT = 1, 128, 10000, 32, 128
NC, NS, L = 2, 16, 16
NW = NC * NS
FPW = OUT // NW
PPW = FPW // 2
OH = OUT // 2
NB = 1024
BLOCKS = [(i * NB, NB) for i in range(N // NB)] + [((N // NB) * NB, N % NB)]


def _pack_rows(even, odd):
    lo = lax.bitcast_convert_type(even.astype(jnp.bfloat16), jnp.uint16)
    hi = lax.bitcast_convert_type(odd.astype(jnp.bfloat16), jnp.uint16)
    return lo.astype(jnp.int32) | (hi.astype(jnp.int32) << 16)


def _tc_tables(x_ref, m1e_ref, m1o_ref, m2e_ref, m2o_ref, be_ref, bo_ref,
               a_ref, c_ref):
    xb = x_ref[...]
    f32 = jnp.float32
    ae = jnp.dot(m1e_ref[...], xb, preferred_element_type=f32) + be_ref[...]
    ao = jnp.dot(m1o_ref[...], xb, preferred_element_type=f32) + bo_ref[...]
    ce = jnp.dot(m2e_ref[...], xb, preferred_element_type=f32)
    co = jnp.dot(m2o_ref[...], xb, preferred_element_type=f32)
    a_ref[...] = _pack_rows(ae, ao)
    c_ref[...] = _pack_rows(ce, co)


def _make_tables(x2, w, b):
    bn = N
    grid = 1
    w1 = w[:, :C] - w[:, C:]
    w2 = w[:, C:]
    mat_spec = pl.BlockSpec((OH, C), lambda i: (0, 0))
    vec_spec = pl.BlockSpec((OH, 1), lambda i: (0, 0))
    return pl.pallas_call(
        _tc_tables,
        grid=(grid,),
        in_specs=[
            pl.BlockSpec((C, bn), lambda i: (0, i)),
            mat_spec, mat_spec, mat_spec, mat_spec, vec_spec, vec_spec,
        ],
        out_specs=[
            pl.BlockSpec((OH, bn), lambda i: (0, i)),
            pl.BlockSpec((OH, bn), lambda i: (0, i)),
        ],
        out_shape=[
            jax.ShapeDtypeStruct((OH, N), jnp.int32),
            jax.ShapeDtypeStruct((OH, N), jnp.int32),
        ],
    )(
        x2, w1[0::2], w1[1::2], w2[0::2], w2[1::2],
        b[0::2].reshape(OH, 1), b[1::2].reshape(OH, 1),
    )


@functools.partial(
    pl.kernel,
    out_type=jax.ShapeDtypeStruct((OUT, N), jnp.float32),
    mesh=plsc.VectorSubcoreMesh(
        core_axis_name="c", subcore_axis_name="s", num_cores=NC, num_subcores=NS
    ),
    compiler_params=pltpu.CompilerParams(
        needs_layout_passes=False, use_tc_tiling_on_sc=False
    ),
    scratch_types=[
        pltpu.VMEM((PPW * N,), jnp.int32),
        pltpu.VMEM((PPW * N,), jnp.int32),
        pltpu.VMEM((K, NB), jnp.int32),
        pltpu.VMEM((K, NB), jnp.int32),
        pltpu.VMEM((FPW, NB), jnp.float32),
        pltpu.VMEM((FPW, NB), jnp.float32),
        pltpu.SemaphoreType.DMA,
        pltpu.SemaphoreType.DMA,
        pltpu.SemaphoreType.DMA,
        pltpu.SemaphoreType.DMA,
    ],
)
def _sc_edge_max(
    a_hbm, c_hbm, ij_hbm, out_hbm,
    a_v, c_v, ij_v0, ij_v1, ob_v0, ob_v1, sem_in0, sem_in1, sem_out0, sem_out1,
):
    wid = lax.axis_index("c") * NS + lax.axis_index("s")
    pltpu.sync_copy(a_hbm.at[wid], a_v)
    pltpu.sync_copy(c_hbm.at[wid], c_v)

    ij_bufs = (ij_v0, ij_v1)
    ob_bufs = (ob_v0, ob_v1)
    sem_in = (sem_in0, sem_in1)
    sem_out = (sem_out0, sem_out1)
    p_off = [jnp.full((L,), p * N, jnp.int32) for p in range(PPW)]

    def start_in(blk):
        bi = blk % 2
        n0, nb = BLOCKS[blk]
        return pltpu.async_copy(
            ij_hbm.at[:, pl.ds(n0, nb)],
            ij_bufs[bi].at[:, pl.ds(0, nb)],
            sem_in[bi],
        )

    pending = {0: start_in(0)}
    out_pending = {}
    for blk in range(len(BLOCKS)):
        bi = blk % 2
        n0, nb = BLOCKS[blk]
        if blk + 1 < len(BLOCKS):
            pending[blk + 1] = start_in(blk + 1)
        pending.pop(blk).wait()
        if blk - 2 in out_pending:
            out_pending.pop(blk - 2).wait()

        ij_b = ij_bufs[bi]
        ob_b = ob_bufs[bi]

        def nb_body(nb_i):
            base = nb_i * L

            def k8_body(k8, accs):
                accs = list(accs)
                for kk in range(8):
                    w = ij_b[k8 * 8 + kk, pl.ds(base, L)]
                    iv = w & 0xFFFF
                    jv = lax.shift_right_logical(w, 16)
                    chain = kk % 2
                    for p in range(PPW):
                        av = plsc.bitcast(
                            plsc.load_gather(a_v, [p_off[p] + iv]), jnp.bfloat16
                        )
                        cv = plsc.bitcast(
                            plsc.load_gather(c_v, [p_off[p] + jv]), jnp.bfloat16
                        )
                        s = 2 * p + chain
                        accs[s] = jnp.maximum(accs[s], av + cv)
                return tuple(accs)

            accs = lax.fori_loop(
                0, K // 8, k8_body,
                tuple(jnp.zeros((2 * L,), jnp.bfloat16) for _ in range(2 * PPW)),
            )
            for p in range(PPW):
                m = plsc.bitcast(
                    jnp.maximum(accs[2 * p], accs[2 * p + 1]), jnp.int32
                )
                ob_b[2 * p, pl.ds(base, L)] = plsc.bitcast(
                    lax.shift_left(m, 16), jnp.float32
                )
                ob_b[2 * p + 1, pl.ds(base, L)] = plsc.bitcast(
                    m & jnp.int32(-65536), jnp.float32
                )

        plsc.parallel_loop(0, nb // L, 1)(nb_body)
        out_pending[blk] = pltpu.async_copy(
            ob_b.at[:, pl.ds(0, nb)],
            out_hbm.at[pl.ds(wid * FPW, FPW), pl.ds(n0, nb)],
            sem_out[bi],
        )
    for h in out_pending.values():
        h.wait()


def kernel(x, edge_index, W, b):
    x2 = x[0, :, :, 0]
    a_t, c_t = _make_tables(x2, W, b)
    a_r = a_t.reshape(NW, PPW * N)
    c_r = c_t.reshape(NW, PPW * N)

    ij = edge_index[1, 0] | (edge_index[0, 0] << 16)
    out = _sc_edge_max(a_r, c_r, ij.T)
    return out.reshape(1, OUT, N, 1)
